# baseline (speedup 1.0000x reference)
import functools

import jax
import jax.numpy as jnp
from jax.experimental import pallas as pl
from jax.experimental.pallas import tpu as pltpu

BM = 200


def _small_matmul_kernel(a_ref, w_ref, o_ref):
    o_ref[:] = jnp.dot(a_ref[:], w_ref[:], preferred_element_type=jnp.float32)


def _adj_pass1_kernel(adj_ref, s_ref, b_ref, w2_ref, o_ref):
    acc = jnp.dot(adj_ref[0], s_ref[:], preferred_element_type=jnp.float32)
    h = jnp.maximum(acc + b_ref[:], 0.0)
    o_ref[:] = jnp.dot(h, w2_ref[:], preferred_element_type=jnp.float32)


def _adj_pass2_kernel(adj_ref, s_ref, b_ref, o_ref):
    acc = jnp.dot(adj_ref[0], s_ref[:], preferred_element_type=jnp.float32)
    g = acc + b_ref[:]
    m = jnp.max(g, axis=1, keepdims=True)
    sh = g - m
    lse = jnp.log(jnp.sum(jnp.exp(sh), axis=1, keepdims=True))
    o_ref[:] = sh - lse


def _small_matmul(a, w):
    n, _ = a.shape
    f = w.shape[1]
    return pl.pallas_call(
        _small_matmul_kernel,
        out_shape=jax.ShapeDtypeStruct((n, f), jnp.float32),
    )(a, w)


def _adj_pass(adj, phase, kernel_body, fout, *small_inputs):
    n = adj.shape[1]
    grid = (n // BM,)
    small_specs = [
        pl.BlockSpec(s.shape, lambda i: tuple(0 for _ in s.shape))
        for s in small_inputs
    ]
    return pl.pallas_call(
        kernel_body,
        grid=grid,
        in_specs=[pl.BlockSpec((1, BM, n), lambda i: (phase, i, 0))] + small_specs,
        out_specs=pl.BlockSpec((BM, fout), lambda i: (i, 0)),
        out_shape=jax.ShapeDtypeStruct((n, fout), jnp.float32),
        compiler_params=pltpu.CompilerParams(
            dimension_semantics=("parallel",),
        ),
    )(adj, *small_inputs)


@jax.jit
def kernel(x, adj, W1, b1, W2, b2):
    nclass = W2.shape[1]
    support1 = _small_matmul(x, W1)
    support2 = _adj_pass(adj, 0, _adj_pass1_kernel, nclass,
                         support1, b1.reshape(1, -1), W2)
    out = _adj_pass(adj, 1, _adj_pass2_kernel, nclass,
                    support2, b2.reshape(1, -1))
    return out


# speedup vs baseline: 1.0781x; 1.0781x over previous
import jax
import jax.numpy as jnp
from jax.experimental import pallas as pl
from jax.experimental.pallas import tpu as pltpu

BM = 400


def _gcn_kernel(adj_ref, x_ref, w1_ref, b1_ref, w2_ref, b2_ref,
                o_ref, s1_ref, s2_ref):
    p = pl.program_id(0)
    i = pl.program_id(1)

    @pl.when((p == 0) & (i == 0))
    def _init():
        s1_ref[:] = jnp.dot(x_ref[:], w1_ref[:], preferred_element_type=jnp.float32)

    @pl.when(p == 0)
    def _layer1():
        h = jnp.maximum(
            jnp.dot(adj_ref[0], s1_ref[:], preferred_element_type=jnp.float32)
            + b1_ref[:], 0.0)
        s2 = jnp.dot(h, w2_ref[:], preferred_element_type=jnp.float32)
        s2_ref[pl.ds(i * BM, BM), :] = s2
        o_ref[:] = s2

    @pl.when(p == 1)
    def _layer2():
        g = jnp.dot(adj_ref[0], s2_ref[:], preferred_element_type=jnp.float32) \
            + b2_ref[:]
        m = jnp.max(g, axis=1, keepdims=True)
        sh = g - m
        lse = jnp.log(jnp.sum(jnp.exp(sh), axis=1, keepdims=True))
        o_ref[:] = sh - lse


@jax.jit
def kernel(x, adj, W1, b1, W2, b2):
    n = adj.shape[1]
    nhid = W1.shape[1]
    nclass = W2.shape[1]
    nb = n // BM
    return pl.pallas_call(
        _gcn_kernel,
        grid=(2, nb),
        in_specs=[
            pl.BlockSpec((1, BM, n), lambda p, i: (p, i, 0)),
            pl.BlockSpec(x.shape, lambda p, i: (0, 0)),
            pl.BlockSpec(W1.shape, lambda p, i: (0, 0)),
            pl.BlockSpec((1, nhid), lambda p, i: (0, 0)),
            pl.BlockSpec(W2.shape, lambda p, i: (0, 0)),
            pl.BlockSpec((1, nclass), lambda p, i: (0, 0)),
        ],
        out_specs=pl.BlockSpec((BM, nclass), lambda p, i: (i, 0)),
        out_shape=jax.ShapeDtypeStruct((n, nclass), jnp.float32),
        scratch_shapes=[
            pltpu.VMEM((n, nhid), jnp.float32),
            pltpu.VMEM((n, nclass), jnp.float32),
        ],
        compiler_params=pltpu.CompilerParams(
            dimension_semantics=("arbitrary", "arbitrary"),
        ),
    )(adj, x, W1, b1.reshape(1, -1), W2, b2.reshape(1, -1))
